# baseline (speedup 1.0000x reference)
import functools

import jax
import jax.numpy as jnp
from jax import lax
from jax.experimental import pallas as pl
from jax.experimental.pallas import tpu as pltpu
from jax.experimental.pallas import tpu_sc as plsc

N = 10000
E = 320000
D = 128
H = 8
C = 16
FF = 256
NGRAPH = 64
NFREQ = 64
EDIM = 16

F32 = jnp.float32

NP = 10112
BN = 1264
NT = 16
B = 128
ET = 20224
NBLK = ET // B
E_PAD = NT * ET
ROWS_T = NP // NT
HD = D // 2
HH = H // 2
WACC = 80


def _row_specs(args, row_args):
    in_specs = []
    for a, rb in zip(args, row_args):
        if rb:
            in_specs.append(
                pl.BlockSpec((BN,) + a.shape[1:],
                             lambda i, _nd=a.ndim: (i,) + (0,) * (_nd - 1)))
        else:
            in_specs.append(
                pl.BlockSpec(a.shape, lambda i, _nd=a.ndim: (0,) * _nd))
    return in_specs


def _pc_rows(body, out_shapes, out_blocks, args, row_args):
    out_specs = tuple(
        pl.BlockSpec(b, (lambda i: (0, i, 0)) if len(b) == 3 else
                     (lambda i: (i, 0)))
        for b in out_blocks)
    res = pl.pallas_call(
        body,
        grid=(NP // BN,),
        in_specs=_row_specs(args, row_args),
        out_specs=out_specs if len(out_shapes) > 1 else out_specs[0],
        out_shape=(tuple(out_shapes) if len(out_shapes) > 1
                   else out_shapes[0]),
    )(*args)
    return res


def _pre_body(freq_ref, npa_ref, win_ref, bin_ref, wemb_ref, bemb_ref,
              inp_ref, x0_ref):
    inp_ref[...] = jnp.dot(freq_ref[...], win_ref[...],
                           preferred_element_type=F32) + bin_ref[...]
    x0_ref[...] = jnp.dot(npa_ref[...], wemb_ref[...],
                          preferred_element_type=F32) + bemb_ref[...]


def _qkv_body(x_ref, inp_ref, wq_ref, bq_ref, wk_ref, bk_ref, wv_ref, bv_ref,
              wskip_ref, bskip_ref,
              xin_ref, q_ref, k_ref, v_ref, skip_ref):
    xin = x_ref[...] + inp_ref[...]
    xin_ref[...] = xin
    for w_ref, b_ref, o_ref in ((wq_ref, bq_ref, q_ref),
                                (wk_ref, bk_ref, k_ref),
                                (wv_ref, bv_ref, v_ref)):
        t = jnp.dot(xin, w_ref[...], preferred_element_type=F32) + b_ref[...]
        o_ref[0] = t[:, :HD]
        o_ref[1] = t[:, HD:]
    skip_ref[...] = jnp.dot(xin, wskip_ref[...],
                            preferred_element_type=F32) + bskip_ref[...]


def _edge_e_body(ea_ref, we_ref, e_ref):
    t = jnp.dot(ea_ref[...], we_ref[...], preferred_element_type=F32)
    e_ref[0] = t[:, :HD]
    e_ref[1] = t[:, HD:]


def _ln(x, g, b, eps=1e-5):
    mu = jnp.mean(x, axis=-1, keepdims=True)
    var = jnp.mean((x - mu) ** 2, axis=-1, keepdims=True)
    return (x - mu) * jax.lax.rsqrt(var + eps) * g + b


def _post_body(acc0_ref, acc1_ref, xin_ref, skip_ref,
               ln1g_ref, ln1b_ref, w1_ref, b1_ref, w2_ref, b2_ref,
               ln2g_ref, ln2b_ref, xout_ref):
    num = jnp.concatenate([acc0_ref[...][:, :HD], acc1_ref[...][:, :HD]],
                          axis=1)
    den = jnp.concatenate([acc0_ref[...][:, HD:HD + HH],
                           acc1_ref[...][:, HD:HD + HH]], axis=1)
    dinv = 1.0 / (den + 1e-16)
    dinv_e = jnp.broadcast_to(dinv[:, :, None], (BN, H, C)).reshape(BN, D)
    conv = num * dinv_e + skip_ref[...]
    x1 = _ln(xin_ref[...] + conv, ln1g_ref[...], ln1b_ref[...])
    h1 = jnp.maximum(
        jnp.dot(x1, w1_ref[...], preferred_element_type=F32) + b1_ref[...],
        0.0)
    x2 = jnp.dot(h1, w2_ref[...], preferred_element_type=F32) + b2_ref[...]
    xout_ref[...] = _ln(x1 + x2, ln2g_ref[...], ln2b_ref[...])


def _pool_body(x_ref, batch_ref, wout_ref, bout_ref, val_ref):
    gids = jax.lax.broadcasted_iota(jnp.int32, (NGRAPH, NP), 0)
    mask = (gids == batch_ref[...]).astype(F32)
    s = jnp.dot(mask, x_ref[...], preferred_element_type=F32)
    cnt = jnp.sum(mask, axis=1, keepdims=True)
    mean = s / jnp.maximum(cnt, 1.0)
    val_ref[...] = jnp.dot(mean, wout_ref[...],
                           preferred_element_type=F32) + bout_ref[...]


def _sc_edge_body(q_hbm, k_hbm, v_hbm, e_hbm, src_hbm, dst_hbm,
                  out0_hbm, out1_hbm,
                  idx_s, idx_d, idx_d2, qrows, krows, vrows, erows, orows,
                  acc, sem):
    cid = lax.axis_index("c")
    sid = lax.axis_index("s")
    zeros16 = jnp.zeros((16,), F32)
    lane = lax.iota(jnp.int32, 16)

    def _zrow(r, carry):
        row = jnp.full((16,), r, jnp.int32)
        for j in range(WACC // 16):
            plsc.store_scatter(orows, [row, lane + j * 16], zeros16)
        return carry
    lax.fori_loop(0, B, _zrow, 0)

    row0 = sid * ROWS_T
    off = 0
    while off < ROWS_T:
        ln = min(B, ROWS_T - off)
        pltpu.sync_copy(orows.at[pl.ds(0, ln)], acc.at[pl.ds(row0 + off, ln)])
        off += ln
    plsc.subcore_barrier()

    tbl_off = cid * NP
    e_off = cid * E_PAD

    def _block(b, carry):
        ebase = sid * ET + b * B
        pltpu.sync_copy(src_hbm.at[pl.ds(ebase, B)], idx_s)
        pltpu.sync_copy(dst_hbm.at[pl.ds(ebase, B)], idx_d)
        for j in range(B // 16):
            sl = pl.ds(j * 16, 16)
            idx_s[sl] = idx_s[sl] + tbl_off
            idx_d2[sl] = idx_d[sl] + tbl_off
        cps = [
            pltpu.async_copy(k_hbm.at[idx_s], krows, sem),
            pltpu.async_copy(v_hbm.at[idx_s], vrows, sem),
            pltpu.async_copy(q_hbm.at[idx_d2], qrows, sem),
            pltpu.async_copy(e_hbm.at[pl.ds(e_off + ebase, B)], erows, sem),
        ]
        for cp in cps:
            cp.wait()

        @plsc.parallel_loop(0, B // 16, unroll=2)
        def _group(g):
            eids = lane + g * 16
            for h in range(HH):
                acc_a = zeros16
                for c in range(C):
                    fcol = jnp.full((16,), h * C + c, jnp.int32)
                    qv = plsc.load_gather(qrows, [eids, fcol])
                    kv = plsc.load_gather(krows, [eids, fcol])
                    ev = plsc.load_gather(erows, [eids, fcol])
                    acc_a = acc_a + qv * (kv + ev)
                s = jnp.exp(acc_a * 0.25)
                for c in range(C):
                    fcol = jnp.full((16,), h * C + c, jnp.int32)
                    vv = plsc.load_gather(vrows, [eids, fcol])
                    ev = plsc.load_gather(erows, [eids, fcol])
                    plsc.store_scatter(orows, [eids, fcol], s * (vv + ev))
                plsc.store_scatter(
                    orows, [eids, jnp.full((16,), HD + h, jnp.int32)], s)
        pltpu.sync_copy(orows, acc.at[idx_d], add=True)
        return carry
    lax.fori_loop(0, NBLK, _block, 0)

    plsc.subcore_barrier()
    off = 0
    while off < ROWS_T:
        ln = min(B, ROWS_T - off)
        sl = pl.ds(row0 + off, ln)

        @pl.when(cid == 0)
        def _():
            pltpu.sync_copy(acc.at[sl], out0_hbm.at[sl])

        @pl.when(cid == 1)
        def _():
            pltpu.sync_copy(acc.at[sl], out1_hbm.at[sl])
        off += ln


@functools.partial(
    pl.kernel,
    out_type=(jax.ShapeDtypeStruct((NP, WACC), F32),
              jax.ShapeDtypeStruct((NP, WACC), F32)),
    mesh=plsc.VectorSubcoreMesh(core_axis_name="c", subcore_axis_name="s",
                                num_cores=2, num_subcores=16),
    compiler_params=pltpu.CompilerParams(use_tc_tiling_on_sc=False,
                                         needs_layout_passes=False),
    scratch_types=[
        pltpu.VMEM((B,), jnp.int32),
        pltpu.VMEM((B,), jnp.int32),
        pltpu.VMEM((B,), jnp.int32),
        pltpu.VMEM((B, HD), F32),
        pltpu.VMEM((B, HD), F32),
        pltpu.VMEM((B, HD), F32),
        pltpu.VMEM((B, HD), F32),
        pltpu.VMEM((B, WACC), F32),
        pltpu.VMEM_SHARED((NP, WACC), F32),
        pltpu.SemaphoreType.DMA,
    ],
)
def _sc_edge(q_hbm, k_hbm, v_hbm, e_hbm, src_hbm, dst_hbm,
             out0_hbm, out1_hbm, *scratch):
    _sc_edge_body(q_hbm, k_hbm, v_hbm, e_hbm, src_hbm, dst_hbm,
                  out0_hbm, out1_hbm, *scratch)


def kernel(freq_alloc, node_power_attn, edge_power_attn, edge_index, batch,
           params):
    src = jnp.concatenate(
        [edge_index[0].astype(jnp.int32),
         jnp.full((E_PAD - E,), N, jnp.int32)])
    dst = jnp.concatenate(
        [edge_index[1].astype(jnp.int32),
         jnp.full((E_PAD - E,), N, jnp.int32)])
    ea_pad = jnp.concatenate(
        [edge_power_attn, jnp.zeros((E_PAD - E, EDIM), F32)])
    freq_pad = jnp.concatenate([freq_alloc, jnp.zeros((NP - N, NFREQ), F32)])
    npa_pad = jnp.concatenate(
        [node_power_attn, jnp.zeros((NP - N, EDIM), F32)])
    batch_pad = jnp.concatenate(
        [batch.astype(jnp.int32), jnp.full((NP - N,), -1, jnp.int32)])

    b2 = lambda b: b.reshape(1, -1)
    inp, x = _pc_rows(
        _pre_body,
        (jax.ShapeDtypeStruct((NP, D), F32), jax.ShapeDtypeStruct((NP, D), F32)),
        ((BN, D), (BN, D)),
        (freq_pad, npa_pad, params['Win'], b2(params['bin']),
         params['Wemb'], b2(params['bemb'])),
        (True, True, False, False, False, False))

    for lp in params['layers']:
        xin, q2, k2, v2, skip = _pc_rows(
            _qkv_body,
            (jax.ShapeDtypeStruct((NP, D), F32),
             jax.ShapeDtypeStruct((2, NP, HD), F32),
             jax.ShapeDtypeStruct((2, NP, HD), F32),
             jax.ShapeDtypeStruct((2, NP, HD), F32),
             jax.ShapeDtypeStruct((NP, D), F32)),
            ((BN, D), (2, BN, HD), (2, BN, HD), (2, BN, HD), (BN, D)),
            (x, inp, lp['Wq'], b2(lp['bq']), lp['Wk'], b2(lp['bk']),
             lp['Wv'], b2(lp['bv']), lp['Wskip'], b2(lp['bskip'])),
            (True, True, False, False, False, False, False, False, False,
             False))

        eb = 32
        e2 = pl.pallas_call(
            _edge_e_body,
            grid=(eb,),
            in_specs=[
                pl.BlockSpec((E_PAD // eb, EDIM), lambda i: (i, 0)),
                pl.BlockSpec((EDIM, D), lambda i: (0, 0)),
            ],
            out_specs=pl.BlockSpec((2, E_PAD // eb, HD), lambda i: (0, i, 0)),
            out_shape=jax.ShapeDtypeStruct((2, E_PAD, HD), F32),
        )(ea_pad, lp['We'])

        acc0, acc1 = _sc_edge(
            q2.reshape(2 * NP, HD), k2.reshape(2 * NP, HD),
            v2.reshape(2 * NP, HD), e2.reshape(2 * E_PAD, HD), src, dst)

        x = _pc_rows(
            _post_body, (jax.ShapeDtypeStruct((NP, D), F32),), ((BN, D),),
            (acc0, acc1, xin, skip,
             b2(lp['ln1_g']), b2(lp['ln1_b']), lp['W1'], b2(lp['b1']),
             lp['W2'], b2(lp['b2']), b2(lp['ln2_g']), b2(lp['ln2_b'])),
            (True, True, True, True, False, False, False, False, False,
             False, False, False))

    val = pl.pallas_call(
        _pool_body,
        out_shape=jax.ShapeDtypeStruct((NGRAPH, 1), F32),
    )(x, batch_pad.reshape(1, NP), params['Wout'], b2(params['bout']))
    return val.reshape(NGRAPH)


# speedup vs baseline: 3.7613x; 3.7613x over previous
import functools

import jax
import jax.numpy as jnp
from jax import lax
from jax.experimental import pallas as pl
from jax.experimental.pallas import tpu as pltpu
from jax.experimental.pallas import tpu_sc as plsc

N = 10000
E = 320000
D = 128
H = 8
C = 16
FF = 256
NGRAPH = 64
NFREQ = 64
EDIM = 16

F32 = jnp.float32

NP = 10112
BN = 1264
NT = 16
B = 128
ET = 20224
NBLK = ET // B
E_PAD = NT * ET
ROWS_T = NP // NT
HD = D // 2
HH = H // 2
WACC = 80


def _row_specs(args, row_args):
    in_specs = []
    for a, rb in zip(args, row_args):
        if rb:
            in_specs.append(
                pl.BlockSpec((BN,) + a.shape[1:],
                             lambda i, _nd=a.ndim: (i,) + (0,) * (_nd - 1)))
        else:
            in_specs.append(
                pl.BlockSpec(a.shape, lambda i, _nd=a.ndim: (0,) * _nd))
    return in_specs


def _pc_rows(body, out_shapes, out_blocks, args, row_args):
    out_specs = tuple(
        pl.BlockSpec(b, (lambda i: (0, i, 0)) if len(b) == 3 else
                     (lambda i: (i, 0)))
        for b in out_blocks)
    res = pl.pallas_call(
        body,
        grid=(NP // BN,),
        in_specs=_row_specs(args, row_args),
        out_specs=out_specs if len(out_shapes) > 1 else out_specs[0],
        out_shape=(tuple(out_shapes) if len(out_shapes) > 1
                   else out_shapes[0]),
    )(*args)
    return res


def _pre_body(freq_ref, npa_ref, win_ref, bin_ref, wemb_ref, bemb_ref,
              inp_ref, x0_ref):
    inp_ref[...] = jnp.dot(freq_ref[...], win_ref[...],
                           preferred_element_type=F32) + bin_ref[...]
    x0_ref[...] = jnp.dot(npa_ref[...], wemb_ref[...],
                          preferred_element_type=F32) + bemb_ref[...]


def _qkv_body(x_ref, inp_ref, wq_ref, bq_ref, wk_ref, bk_ref, wv_ref, bv_ref,
              wskip_ref, bskip_ref,
              xin_ref, q_ref, k_ref, v_ref, skip_ref):
    xin = x_ref[...] + inp_ref[...]
    xin_ref[...] = xin
    for w_ref, b_ref, o_ref in ((wq_ref, bq_ref, q_ref),
                                (wk_ref, bk_ref, k_ref),
                                (wv_ref, bv_ref, v_ref)):
        t = jnp.dot(xin, w_ref[...], preferred_element_type=F32) + b_ref[...]
        o_ref[0] = t[:, :HD]
        o_ref[1] = t[:, HD:]
    skip_ref[...] = jnp.dot(xin, wskip_ref[...],
                            preferred_element_type=F32) + bskip_ref[...]


def _edge_e_body(ea_ref, we_ref, e_ref):
    t = jnp.dot(ea_ref[...], we_ref[...], preferred_element_type=F32)
    e_ref[0] = t[:, :HD]
    e_ref[1] = t[:, HD:]


def _ln(x, g, b, eps=1e-5):
    mu = jnp.mean(x, axis=-1, keepdims=True)
    var = jnp.mean((x - mu) ** 2, axis=-1, keepdims=True)
    return (x - mu) * jax.lax.rsqrt(var + eps) * g + b


def _post_body(acc0_ref, acc1_ref, xin_ref, skip_ref,
               ln1g_ref, ln1b_ref, w1_ref, b1_ref, w2_ref, b2_ref,
               ln2g_ref, ln2b_ref, xout_ref):
    num = jnp.concatenate([acc0_ref[...][:, :HD], acc1_ref[...][:, :HD]],
                          axis=1)
    den = jnp.concatenate([acc0_ref[...][:, HD:HD + HH],
                           acc1_ref[...][:, HD:HD + HH]], axis=1)
    dinv = 1.0 / (den + 1e-16)
    dinv_e = jnp.broadcast_to(dinv[:, :, None], (BN, H, C)).reshape(BN, D)
    conv = num * dinv_e + skip_ref[...]
    x1 = _ln(xin_ref[...] + conv, ln1g_ref[...], ln1b_ref[...])
    h1 = jnp.maximum(
        jnp.dot(x1, w1_ref[...], preferred_element_type=F32) + b1_ref[...],
        0.0)
    x2 = jnp.dot(h1, w2_ref[...], preferred_element_type=F32) + b2_ref[...]
    xout_ref[...] = _ln(x1 + x2, ln2g_ref[...], ln2b_ref[...])


def _pool_body(x_ref, batch_ref, wout_ref, bout_ref, val_ref):
    gids = jax.lax.broadcasted_iota(jnp.int32, (NGRAPH, NP), 0)
    mask = (gids == batch_ref[...]).astype(F32)
    s = jnp.dot(mask, x_ref[...], preferred_element_type=F32)
    cnt = jnp.sum(mask, axis=1, keepdims=True)
    mean = s / jnp.maximum(cnt, 1.0)
    val_ref[...] = jnp.dot(mean, wout_ref[...],
                           preferred_element_type=F32) + bout_ref[...]


def _sc_edge_body(q_hbm, k_hbm, v_hbm, e_hbm, src_hbm, dst_hbm,
                  out0_hbm, out1_hbm,
                  idx_s, idx_d, idx_d2, qrows, krows, vrows, erows, orows,
                  acc, sem):
    cid = lax.axis_index("c")
    sid = lax.axis_index("s")
    zeros16 = jnp.zeros((16,), F32)
    lane = lax.iota(jnp.int32, 16)

    def _zrow(r, carry):
        row = jnp.full((16,), r, jnp.int32)
        for j in range(WACC // 16):
            plsc.store_scatter(orows, [row, lane + j * 16], zeros16)
        return carry
    lax.fori_loop(0, B, _zrow, 0)

    row0 = sid * ROWS_T
    off = 0
    while off < ROWS_T:
        ln = min(B, ROWS_T - off)
        pltpu.sync_copy(orows.at[pl.ds(0, ln)], acc.at[pl.ds(row0 + off, ln)])
        off += ln
    plsc.subcore_barrier()

    tbl_off = cid * NP
    e_off = cid * E_PAD

    def _block(b, carry):
        ebase = sid * ET + b * B
        pltpu.sync_copy(src_hbm.at[pl.ds(ebase, B)], idx_s)
        pltpu.sync_copy(dst_hbm.at[pl.ds(ebase, B)], idx_d)
        for j in range(B // 16):
            sl = pl.ds(j * 16, 16)
            idx_s[sl] = idx_s[sl] + tbl_off
            idx_d2[sl] = idx_d[sl] + tbl_off
        cps = [
            pltpu.async_copy(k_hbm.at[idx_s], krows, sem),
            pltpu.async_copy(v_hbm.at[idx_s], vrows, sem),
            pltpu.async_copy(q_hbm.at[idx_d2], qrows, sem),
            pltpu.async_copy(e_hbm.at[pl.ds(e_off + ebase, B)], erows, sem),
        ]
        for cp in cps:
            cp.wait()

        @plsc.parallel_loop(0, B, unroll=4)
        def _edge(i):
            den = zeros16
            for h in range(HH):
                hs = pl.ds(h * C, C)
                eh = erows[i, hs]
                t = qrows[i, hs] * (krows[i, hs] + eh)
                a = jnp.sum(t) * 0.25
                s = jnp.exp(jnp.full((16,), a, F32))
                orows[i, hs] = s * (vrows[i, hs] + eh)
                den = jnp.where(lane == h, s, den)
            orows[i, pl.ds(HD, 16)] = den
        pltpu.sync_copy(orows, acc.at[idx_d], add=True)
        return carry
    lax.fori_loop(0, NBLK, _block, 0)

    plsc.subcore_barrier()
    off = 0
    while off < ROWS_T:
        ln = min(B, ROWS_T - off)
        sl = pl.ds(row0 + off, ln)

        @pl.when(cid == 0)
        def _():
            pltpu.sync_copy(acc.at[sl], out0_hbm.at[sl])

        @pl.when(cid == 1)
        def _():
            pltpu.sync_copy(acc.at[sl], out1_hbm.at[sl])
        off += ln


@functools.partial(
    pl.kernel,
    out_type=(jax.ShapeDtypeStruct((NP, WACC), F32),
              jax.ShapeDtypeStruct((NP, WACC), F32)),
    mesh=plsc.VectorSubcoreMesh(core_axis_name="c", subcore_axis_name="s",
                                num_cores=2, num_subcores=16),
    compiler_params=pltpu.CompilerParams(use_tc_tiling_on_sc=False,
                                         needs_layout_passes=False),
    scratch_types=[
        pltpu.VMEM((B,), jnp.int32),
        pltpu.VMEM((B,), jnp.int32),
        pltpu.VMEM((B,), jnp.int32),
        pltpu.VMEM((B, HD), F32),
        pltpu.VMEM((B, HD), F32),
        pltpu.VMEM((B, HD), F32),
        pltpu.VMEM((B, HD), F32),
        pltpu.VMEM((B, WACC), F32),
        pltpu.VMEM_SHARED((NP, WACC), F32),
        pltpu.SemaphoreType.DMA,
    ],
)
def _sc_edge(q_hbm, k_hbm, v_hbm, e_hbm, src_hbm, dst_hbm,
             out0_hbm, out1_hbm, *scratch):
    _sc_edge_body(q_hbm, k_hbm, v_hbm, e_hbm, src_hbm, dst_hbm,
                  out0_hbm, out1_hbm, *scratch)


def kernel(freq_alloc, node_power_attn, edge_power_attn, edge_index, batch,
           params):
    src = jnp.concatenate(
        [edge_index[0].astype(jnp.int32),
         jnp.full((E_PAD - E,), N, jnp.int32)])
    dst = jnp.concatenate(
        [edge_index[1].astype(jnp.int32),
         jnp.full((E_PAD - E,), N, jnp.int32)])
    ea_pad = jnp.concatenate(
        [edge_power_attn, jnp.zeros((E_PAD - E, EDIM), F32)])
    freq_pad = jnp.concatenate([freq_alloc, jnp.zeros((NP - N, NFREQ), F32)])
    npa_pad = jnp.concatenate(
        [node_power_attn, jnp.zeros((NP - N, EDIM), F32)])
    batch_pad = jnp.concatenate(
        [batch.astype(jnp.int32), jnp.full((NP - N,), -1, jnp.int32)])

    b2 = lambda b: b.reshape(1, -1)
    inp, x = _pc_rows(
        _pre_body,
        (jax.ShapeDtypeStruct((NP, D), F32), jax.ShapeDtypeStruct((NP, D), F32)),
        ((BN, D), (BN, D)),
        (freq_pad, npa_pad, params['Win'], b2(params['bin']),
         params['Wemb'], b2(params['bemb'])),
        (True, True, False, False, False, False))

    for lp in params['layers']:
        xin, q2, k2, v2, skip = _pc_rows(
            _qkv_body,
            (jax.ShapeDtypeStruct((NP, D), F32),
             jax.ShapeDtypeStruct((2, NP, HD), F32),
             jax.ShapeDtypeStruct((2, NP, HD), F32),
             jax.ShapeDtypeStruct((2, NP, HD), F32),
             jax.ShapeDtypeStruct((NP, D), F32)),
            ((BN, D), (2, BN, HD), (2, BN, HD), (2, BN, HD), (BN, D)),
            (x, inp, lp['Wq'], b2(lp['bq']), lp['Wk'], b2(lp['bk']),
             lp['Wv'], b2(lp['bv']), lp['Wskip'], b2(lp['bskip'])),
            (True, True, False, False, False, False, False, False, False,
             False))

        eb = 32
        e2 = pl.pallas_call(
            _edge_e_body,
            grid=(eb,),
            in_specs=[
                pl.BlockSpec((E_PAD // eb, EDIM), lambda i: (i, 0)),
                pl.BlockSpec((EDIM, D), lambda i: (0, 0)),
            ],
            out_specs=pl.BlockSpec((2, E_PAD // eb, HD), lambda i: (0, i, 0)),
            out_shape=jax.ShapeDtypeStruct((2, E_PAD, HD), F32),
        )(ea_pad, lp['We'])

        acc0, acc1 = _sc_edge(
            q2.reshape(2 * NP, HD), k2.reshape(2 * NP, HD),
            v2.reshape(2 * NP, HD), e2.reshape(2 * E_PAD, HD), src, dst)

        x = _pc_rows(
            _post_body, (jax.ShapeDtypeStruct((NP, D), F32),), ((BN, D),),
            (acc0, acc1, xin, skip,
             b2(lp['ln1_g']), b2(lp['ln1_b']), lp['W1'], b2(lp['b1']),
             lp['W2'], b2(lp['b2']), b2(lp['ln2_g']), b2(lp['ln2_b'])),
            (True, True, True, True, False, False, False, False, False,
             False, False, False))

    val = pl.pallas_call(
        _pool_body,
        out_shape=jax.ShapeDtypeStruct((NGRAPH, 1), F32),
    )(x, batch_pad.reshape(1, NP), params['Wout'], b2(params['bout']))
    return val.reshape(NGRAPH)


# speedup vs baseline: 4.3779x; 1.1639x over previous
import functools

import jax
import jax.numpy as jnp
from jax import lax
from jax.experimental import pallas as pl
from jax.experimental.pallas import tpu as pltpu
from jax.experimental.pallas import tpu_sc as plsc

N = 10000
E = 320000
D = 128
H = 8
C = 16
FF = 256
NGRAPH = 64
NFREQ = 64
EDIM = 16

F32 = jnp.float32

NP = 10112
BN = 1264
NT = 16
B = 128
ET = 20224
NBLK = ET // B
E_PAD = NT * ET
ROWS_T = NP // NT
HD = D // 2
HH = H // 2
WACC = 80


def _row_specs(args, row_args):
    in_specs = []
    for a, rb in zip(args, row_args):
        if rb:
            in_specs.append(
                pl.BlockSpec((BN,) + a.shape[1:],
                             lambda i, _nd=a.ndim: (i,) + (0,) * (_nd - 1)))
        else:
            in_specs.append(
                pl.BlockSpec(a.shape, lambda i, _nd=a.ndim: (0,) * _nd))
    return in_specs


def _pc_rows(body, out_shapes, out_blocks, args, row_args):
    out_specs = tuple(
        pl.BlockSpec(b, (lambda i: (0, i, 0)) if len(b) == 3 else
                     (lambda i: (i, 0)))
        for b in out_blocks)
    res = pl.pallas_call(
        body,
        grid=(NP // BN,),
        in_specs=_row_specs(args, row_args),
        out_specs=out_specs if len(out_shapes) > 1 else out_specs[0],
        out_shape=(tuple(out_shapes) if len(out_shapes) > 1
                   else out_shapes[0]),
    )(*args)
    return res


def _pre_body(freq_ref, npa_ref, win_ref, bin_ref, wemb_ref, bemb_ref,
              inp_ref, x0_ref):
    inp_ref[...] = jnp.dot(freq_ref[...], win_ref[...],
                           preferred_element_type=F32) + bin_ref[...]
    x0_ref[...] = jnp.dot(npa_ref[...], wemb_ref[...],
                          preferred_element_type=F32) + bemb_ref[...]


def _qkv_body(x_ref, inp_ref, wq_ref, bq_ref, wk_ref, bk_ref, wv_ref, bv_ref,
              wskip_ref, bskip_ref,
              xin_ref, q_ref, kv_ref, skip_ref):
    xin = x_ref[...] + inp_ref[...]
    xin_ref[...] = xin
    q = (jnp.dot(xin, wq_ref[...], preferred_element_type=F32)
         + bq_ref[...]) * 0.25
    q_ref[0] = q[:, :HD]
    q_ref[1] = q[:, HD:]
    k = jnp.dot(xin, wk_ref[...], preferred_element_type=F32) + bk_ref[...]
    v = jnp.dot(xin, wv_ref[...], preferred_element_type=F32) + bv_ref[...]
    kv_ref[0] = jnp.concatenate([k[:, :HD], v[:, :HD]], axis=1)
    kv_ref[1] = jnp.concatenate([k[:, HD:], v[:, HD:]], axis=1)
    skip_ref[...] = jnp.dot(xin, wskip_ref[...],
                            preferred_element_type=F32) + bskip_ref[...]


def _edge_e_body(ea_ref, we_ref, e_ref):
    t = jnp.dot(ea_ref[...], we_ref[...], preferred_element_type=F32)
    e_ref[0] = t[:, :HD]
    e_ref[1] = t[:, HD:]


def _ln(x, g, b, eps=1e-5):
    mu = jnp.mean(x, axis=-1, keepdims=True)
    var = jnp.mean((x - mu) ** 2, axis=-1, keepdims=True)
    return (x - mu) * jax.lax.rsqrt(var + eps) * g + b


def _post_body(acc0_ref, acc1_ref, xin_ref, skip_ref,
               ln1g_ref, ln1b_ref, w1_ref, b1_ref, w2_ref, b2_ref,
               ln2g_ref, ln2b_ref, xout_ref):
    num = jnp.concatenate([acc0_ref[...][:, :HD], acc1_ref[...][:, :HD]],
                          axis=1)
    den = jnp.concatenate([acc0_ref[...][:, HD:HD + HH],
                           acc1_ref[...][:, HD:HD + HH]], axis=1)
    dinv = 1.0 / (den + 1e-16)
    dinv_e = jnp.broadcast_to(dinv[:, :, None], (BN, H, C)).reshape(BN, D)
    conv = num * dinv_e + skip_ref[...]
    x1 = _ln(xin_ref[...] + conv, ln1g_ref[...], ln1b_ref[...])
    h1 = jnp.maximum(
        jnp.dot(x1, w1_ref[...], preferred_element_type=F32) + b1_ref[...],
        0.0)
    x2 = jnp.dot(h1, w2_ref[...], preferred_element_type=F32) + b2_ref[...]
    xout_ref[...] = _ln(x1 + x2, ln2g_ref[...], ln2b_ref[...])


def _pool_body(x_ref, batch_ref, wout_ref, bout_ref, val_ref):
    gids = jax.lax.broadcasted_iota(jnp.int32, (NGRAPH, NP), 0)
    mask = (gids == batch_ref[...]).astype(F32)
    s = jnp.dot(mask, x_ref[...], preferred_element_type=F32)
    cnt = jnp.sum(mask, axis=1, keepdims=True)
    mean = s / jnp.maximum(cnt, 1.0)
    val_ref[...] = jnp.dot(mean, wout_ref[...],
                           preferred_element_type=F32) + bout_ref[...]


def _sc_edge_body(q_hbm, kv_hbm, e_hbm, sd_hbm, out0_hbm, out1_hbm,
                  sdb0, sdb1, isrc0, isrc1, idsts0, idsts1, idstr0, idstr1,
                  iscr0, iscr1, qr0, qr1, kvr0, kvr1, er0, org0, org1,
                  acc, sem_sd0, sem_sd1, sem_g0, sem_g1, sem_sc0, sem_sc1):
    cid = lax.axis_index("c")
    sid = lax.axis_index("s")
    zeros16 = jnp.zeros((16,), F32)
    lane = lax.iota(jnp.int32, 16)

    sdb = (sdb0, sdb1)
    isrc = (isrc0, isrc1)
    idsts = (idsts0, idsts1)
    idstr = (idstr0, idstr1)
    iscr = (iscr0, iscr1)
    qr = (qr0, qr1)
    kvr = (kvr0, kvr1)
    org = (org0, org1)
    sem_sd = (sem_sd0, sem_sd1)
    sem_g = (sem_g0, sem_g1)
    sem_sc = (sem_sc0, sem_sc1)

    tbl_off = cid * NP
    e_off = cid * E_PAD
    base = sid * ET

    def _zrow2(r, carry):
        for j in range(WACC // 16):
            org0[r, pl.ds(j * 16, 16)] = zeros16
        return carry
    lax.fori_loop(0, B, _zrow2, 0)

    row0 = sid * ROWS_T
    off = 0
    while off < ROWS_T:
        ln = min(B, ROWS_T - off)
        pltpu.sync_copy(org0.at[pl.ds(0, ln)], acc.at[pl.ds(row0 + off, ln)])
        off += ln
    plsc.subcore_barrier()

    def sd_src(b):
        return sd_hbm.at[:, pl.ds(base + b * B, B)]

    def issue_sd(t, b):
        pltpu.async_copy(sd_src(b), sdb[t], sem_sd[t])

    def wait_sd(t):
        pltpu.make_async_copy(sd_src(0), sdb[t], sem_sd[t]).wait()

    def build_idx(t):
        for j in range(B // 16):
            sl = pl.ds(j * 16, 16)
            s_ = sdb[t][0, sl]
            d_ = sdb[t][1, sl]
            isrc[t][sl] = s_ + tbl_off
            idsts[t][sl] = d_ + tbl_off
            idstr[t][sl] = d_

    def issue_g(t, b):
        pltpu.async_copy(kv_hbm.at[isrc[t]], kvr[t], sem_g[t])
        pltpu.async_copy(q_hbm.at[idsts[t]], qr[t], sem_g[t])

    def wait_g(t):
        pltpu.make_async_copy(kv_hbm.at[pl.ds(0, B)], kvr[t],
                              sem_g[t]).wait()
        pltpu.make_async_copy(q_hbm.at[pl.ds(0, B)], qr[t], sem_g[t]).wait()

    def compute(t, b):
        pltpu.sync_copy(e_hbm.at[pl.ds(e_off + base + b * B, B)], er0)
        qrt, kvrt, ert, orgt = qr[t], kvr[t], er0, org[t]

        @plsc.parallel_loop(0, B, unroll=4)
        def _edge(i):
            den = zeros16
            for h in range(HH):
                hs = pl.ds(h * C, C)
                eh = ert[i, hs]
                t_ = qrt[i, hs] * (kvrt[i, hs] + eh)
                a = jnp.sum(t_)
                s = jnp.exp(jnp.full((16,), a, F32))
                orgt[i, hs] = s * (kvrt[i, pl.ds(HD + h * C, C)] + eh)
                den = jnp.where(lane == h, s, den)
            orgt[i, pl.ds(HD, 16)] = den

    def issue_sc(t):
        for j in range(B // 16):
            sl = pl.ds(j * 16, 16)
            iscr[t][sl] = idstr[t][sl]
        pltpu.async_copy(org[t], acc.at[iscr[t]], sem_sc[t], add=True)

    def wait_sc(t):
        pltpu.make_async_copy(org[t], acc.at[iscr[t]], sem_sc[t]).wait()

    issue_sd(0, 0)
    issue_sd(1, 1)
    wait_sd(0)
    build_idx(0)
    issue_g(0, 0)

    def _pair(i, carry):
        b0 = 2 * i
        b1 = 2 * i + 1
        b2 = 2 * i + 2
        b3 = 2 * i + 3
        wait_sd(1)
        build_idx(1)
        issue_g(1, b1)

        @pl.when(b2 < NBLK)
        def _():
            issue_sd(0, b2)
        wait_g(0)

        @pl.when(i > 0)
        def _():
            wait_sc(0)
        compute(0, b0)
        issue_sc(0)

        @pl.when(b2 < NBLK)
        def _():
            wait_sd(0)
            build_idx(0)
            issue_g(0, b2)

        @pl.when(b3 < NBLK)
        def _():
            issue_sd(1, b3)
        wait_g(1)

        @pl.when(i > 0)
        def _():
            wait_sc(1)
        compute(1, b1)
        issue_sc(1)
        return carry
    lax.fori_loop(0, NBLK // 2, _pair, 0)
    wait_sc(0)
    wait_sc(1)

    plsc.subcore_barrier()
    off = 0
    while off < ROWS_T:
        ln = min(B, ROWS_T - off)
        sl = pl.ds(row0 + off, ln)

        @pl.when(cid == 0)
        def _():
            pltpu.sync_copy(acc.at[sl], out0_hbm.at[sl])

        @pl.when(cid == 1)
        def _():
            pltpu.sync_copy(acc.at[sl], out1_hbm.at[sl])
        off += ln


@functools.partial(
    pl.kernel,
    out_type=(jax.ShapeDtypeStruct((NP, WACC), F32),
              jax.ShapeDtypeStruct((NP, WACC), F32)),
    mesh=plsc.VectorSubcoreMesh(core_axis_name="c", subcore_axis_name="s",
                                num_cores=2, num_subcores=16),
    compiler_params=pltpu.CompilerParams(use_tc_tiling_on_sc=False,
                                         needs_layout_passes=False),
    scratch_types=[
        pltpu.VMEM((2, B), jnp.int32),
        pltpu.VMEM((2, B), jnp.int32),
        pltpu.VMEM((B,), jnp.int32),
        pltpu.VMEM((B,), jnp.int32),
        pltpu.VMEM((B,), jnp.int32),
        pltpu.VMEM((B,), jnp.int32),
        pltpu.VMEM((B,), jnp.int32),
        pltpu.VMEM((B,), jnp.int32),
        pltpu.VMEM((B,), jnp.int32),
        pltpu.VMEM((B,), jnp.int32),
        pltpu.VMEM((B, HD), F32),
        pltpu.VMEM((B, HD), F32),
        pltpu.VMEM((B, D), F32),
        pltpu.VMEM((B, D), F32),
        pltpu.VMEM((B, HD), F32),
        pltpu.VMEM((B, WACC), F32),
        pltpu.VMEM((B, WACC), F32),
        pltpu.VMEM_SHARED((NP, WACC), F32),
        pltpu.SemaphoreType.DMA,
        pltpu.SemaphoreType.DMA,
        pltpu.SemaphoreType.DMA,
        pltpu.SemaphoreType.DMA,
        pltpu.SemaphoreType.DMA,
        pltpu.SemaphoreType.DMA,
    ],
)
def _sc_edge(q_hbm, kv_hbm, e_hbm, sd_hbm, out0_hbm, out1_hbm, *scratch):
    _sc_edge_body(q_hbm, kv_hbm, e_hbm, sd_hbm, out0_hbm, out1_hbm, *scratch)


def kernel(freq_alloc, node_power_attn, edge_power_attn, edge_index, batch,
           params):
    sd = jnp.concatenate(
        [edge_index.astype(jnp.int32),
         jnp.full((2, E_PAD - E), N, jnp.int32)], axis=1)
    ea_pad = jnp.concatenate(
        [edge_power_attn, jnp.zeros((E_PAD - E, EDIM), F32)])
    freq_pad = jnp.concatenate([freq_alloc, jnp.zeros((NP - N, NFREQ), F32)])
    npa_pad = jnp.concatenate(
        [node_power_attn, jnp.zeros((NP - N, EDIM), F32)])
    batch_pad = jnp.concatenate(
        [batch.astype(jnp.int32), jnp.full((NP - N,), -1, jnp.int32)])

    b2 = lambda b: b.reshape(1, -1)
    inp, x = _pc_rows(
        _pre_body,
        (jax.ShapeDtypeStruct((NP, D), F32), jax.ShapeDtypeStruct((NP, D), F32)),
        ((BN, D), (BN, D)),
        (freq_pad, npa_pad, params['Win'], b2(params['bin']),
         params['Wemb'], b2(params['bemb'])),
        (True, True, False, False, False, False))

    for lp in params['layers']:
        xin, q2, kv2, skip = _pc_rows(
            _qkv_body,
            (jax.ShapeDtypeStruct((NP, D), F32),
             jax.ShapeDtypeStruct((2, NP, HD), F32),
             jax.ShapeDtypeStruct((2, NP, D), F32),
             jax.ShapeDtypeStruct((NP, D), F32)),
            ((BN, D), (2, BN, HD), (2, BN, D), (BN, D)),
            (x, inp, lp['Wq'], b2(lp['bq']), lp['Wk'], b2(lp['bk']),
             lp['Wv'], b2(lp['bv']), lp['Wskip'], b2(lp['bskip'])),
            (True, True, False, False, False, False, False, False, False,
             False))

        eb = 32
        e2 = pl.pallas_call(
            _edge_e_body,
            grid=(eb,),
            in_specs=[
                pl.BlockSpec((E_PAD // eb, EDIM), lambda i: (i, 0)),
                pl.BlockSpec((EDIM, D), lambda i: (0, 0)),
            ],
            out_specs=pl.BlockSpec((2, E_PAD // eb, HD), lambda i: (0, i, 0)),
            out_shape=jax.ShapeDtypeStruct((2, E_PAD, HD), F32),
        )(ea_pad, lp['We'])

        acc0, acc1 = _sc_edge(
            q2.reshape(2 * NP, HD), kv2.reshape(2 * NP, D),
            e2.reshape(2 * E_PAD, HD), sd)

        x = _pc_rows(
            _post_body, (jax.ShapeDtypeStruct((NP, D), F32),), ((BN, D),),
            (acc0, acc1, xin, skip,
             b2(lp['ln1_g']), b2(lp['ln1_b']), lp['W1'], b2(lp['b1']),
             lp['W2'], b2(lp['b2']), b2(lp['ln2_g']), b2(lp['ln2_b'])),
            (True, True, True, True, False, False, False, False, False,
             False, False, False))

    val = pl.pallas_call(
        _pool_body,
        out_shape=jax.ShapeDtypeStruct((NGRAPH, 1), F32),
    )(x, batch_pad.reshape(1, NP), params['Wout'], b2(params['bout']))
    return val.reshape(NGRAPH)
